# baseline (speedup 1.0000x reference)
import functools

import jax
import jax.numpy as jnp
from jax import lax
from jax.experimental import pallas as pl
from jax.experimental.pallas import tpu as pltpu

_BT = 128
_RB = 512
_LANES = 128


def _router_body(x_ref, wg_ref, bgg_ref, eid_ref):
    logits = jnp.dot(x_ref[...], wg_ref[...], preferred_element_type=jnp.float32)
    logits = logits + bgg_ref[...]
    mx = jnp.max(logits, axis=1, keepdims=True)
    ids = lax.broadcasted_iota(jnp.int32, logits.shape, 1)
    eid = jnp.min(jnp.where(logits == mx, ids, _LANES), axis=1)
    eid_ref[0, 0, :] = eid


def _mlp_body(be_ref, x_ref, w1_ref, b1_ref, w2_ref, b2_ref, out_ref):
    h = lax.dot_general(
        x_ref[...], w1_ref[0],
        (((1,), (1,)), ((), ())),
        preferred_element_type=jnp.float32,
    )
    h = jnp.maximum(h + b1_ref[0], 0.0)
    out = jnp.sum(h * w2_ref[0], axis=1) + b2_ref[0, 0, 0]
    out_ref[0, 0, :] = out


def kernel(x, Wg, bg, W1, b1, W2, b2):
    B, S, D = x.shape
    E = Wg.shape[0]
    T = B * S
    x_flat = x.reshape(T, D)

    g = jax.random.gumbel(jax.random.key(42), (T, E), jnp.float32)
    bgg = jnp.full((T, _LANES), -1e30, jnp.float32).at[:, :E].set(bg[None, :] + g)
    wgp = jnp.zeros((D, _LANES), jnp.float32).at[:, :E].set(Wg.T)

    n_rb = T // _RB
    eid = pl.pallas_call(
        _router_body,
        grid=(n_rb,),
        in_specs=[
            pl.BlockSpec((_RB, D), lambda i: (i, 0)),
            pl.BlockSpec((D, _LANES), lambda i: (0, 0)),
            pl.BlockSpec((_RB, _LANES), lambda i: (i, 0)),
        ],
        out_specs=pl.BlockSpec((1, 1, _RB), lambda i: (i, 0, 0)),
        out_shape=jax.ShapeDtypeStruct((n_rb, 1, _RB), jnp.int32),
    )(x_flat, wgp, bgg).reshape(T)

    onehot = (eid[:, None] == jnp.arange(E, dtype=jnp.int32)[None, :]).astype(jnp.int32)
    csum = jnp.cumsum(onehot, axis=0)
    counts = csum[-1]
    rank = jnp.take_along_axis(csum, eid[:, None], axis=1)[:, 0] - 1
    padded = ((counts + _BT - 1) // _BT) * _BT
    p_end = jnp.cumsum(padded)
    dest = (p_end - padded)[eid] + rank

    n_mb = (T + E * (_BT - 1) + _BT - 1) // _BT
    tp = n_mb * _BT
    starts = jnp.arange(n_mb, dtype=jnp.int32) * _BT
    block_expert = jnp.minimum(
        jnp.sum((starts[:, None] >= p_end[None, :]).astype(jnp.int32), axis=1),
        E - 1,
    ).astype(jnp.int32)

    x_pad = jnp.zeros((tp, D), x.dtype).at[dest].set(x_flat)

    out_pad = pl.pallas_call(
        _mlp_body,
        grid_spec=pltpu.PrefetchScalarGridSpec(
            num_scalar_prefetch=1,
            grid=(n_mb,),
            in_specs=[
                pl.BlockSpec((_BT, D), lambda i, be: (i, 0)),
                pl.BlockSpec((1, D, D), lambda i, be: (be[i], 0, 0)),
                pl.BlockSpec((1, 1, D), lambda i, be: (be[i], 0, 0)),
                pl.BlockSpec((1, 1, D), lambda i, be: (be[i], 0, 0)),
                pl.BlockSpec((1, 1, 1), lambda i, be: (be[i], 0, 0)),
            ],
            out_specs=pl.BlockSpec((1, 1, _BT), lambda i, be: (i, 0, 0)),
        ),
        out_shape=jax.ShapeDtypeStruct((n_mb, 1, _BT), jnp.float32),
    )(block_expert, x_pad, W1, b1.reshape(E, 1, D), W2.reshape(E, 1, D),
      b2.reshape(E, 1, 1))

    final = out_pad.reshape(tp)[dest]
    return final.reshape(B, S, 1)
